# baseline (speedup 1.0000x reference)
import functools

import jax
import jax.numpy as jnp
from jax.experimental import pallas as pl


N_TILE = 2000


def _transform_body(relu, mean_ref, x_ref, wl_ref, wr_ref, b_ref, o_ref):
    acc = (
        jnp.dot(mean_ref[...], wl_ref[...], preferred_element_type=jnp.float32)
        + jnp.dot(x_ref[...], wr_ref[...], preferred_element_type=jnp.float32)
        + b_ref[...]
    )
    o_ref[...] = jnp.maximum(acc, 0.0) if relu else acc


def _transform(mean, x, W_l, b, W_r, relu):
    n, d = mean.shape
    h = W_l.shape[1]
    grid = (n // N_TILE,)
    return pl.pallas_call(
        functools.partial(_transform_body, relu),
        grid=grid,
        in_specs=[
            pl.BlockSpec((N_TILE, d), lambda i: (i, 0)),
            pl.BlockSpec((N_TILE, d), lambda i: (i, 0)),
            pl.BlockSpec((d, h), lambda i: (0, 0)),
            pl.BlockSpec((d, h), lambda i: (0, 0)),
            pl.BlockSpec((1, h), lambda i: (0, 0)),
        ],
        out_specs=pl.BlockSpec((N_TILE, h), lambda i: (i, 0)),
        out_shape=jax.ShapeDtypeStruct((n, h), jnp.float32),
    )(mean, x, W_l, W_r, b.reshape(1, h))


def _mean_agg(x_src, edge_index, n_dst):
    src = edge_index[0].astype(jnp.int32)
    dst = edge_index[1].astype(jnp.int32)
    msg = jnp.take(x_src, src, axis=0)
    agg = jax.ops.segment_sum(msg, dst, num_segments=n_dst)
    cnt = jax.ops.segment_sum(
        jnp.ones((edge_index.shape[1],), dtype=x_src.dtype), dst, num_segments=n_dst
    )
    return agg / jnp.maximum(cnt, 1.0)[:, None]


def kernel(x_user, x_item, edge_index_u2i, edge_index_i2u, batch_user, batch_item,
           W1_ui_l, b1_ui_l, W1_ui_r, W1_iu_l, b1_iu_l, W1_iu_r,
           W2_ui_l, b2_ui_l, W2_ui_r, W2_iu_l, b2_iu_l, W2_iu_r,
           W_lin1, b_lin1, W_lin2, b_lin2):
    n_user = x_user.shape[0]
    n_item = x_item.shape[0]
    B = 64

    mean_item = _mean_agg(x_user, edge_index_u2i, n_item)
    mean_user = _mean_agg(x_item, edge_index_i2u, n_user)
    h_item = _transform(mean_item, x_item, W1_ui_l, b1_ui_l, W1_ui_r, True)
    h_user = _transform(mean_user, x_user, W1_iu_l, b1_iu_l, W1_iu_r, True)

    mean_item2 = _mean_agg(h_user, edge_index_u2i, n_item)
    mean_user2 = _mean_agg(h_item, edge_index_i2u, n_user)
    h_item2 = _transform(mean_item2, h_item, W2_ui_l, b2_ui_l, W2_ui_r, False)
    h_user2 = _transform(mean_user2, h_user, W2_iu_l, b2_iu_l, W2_iu_r, False)

    def pool(x, batch):
        s = jax.ops.segment_sum(x, batch.astype(jnp.int32), num_segments=B)
        c = jax.ops.segment_sum(
            jnp.ones((x.shape[0],), dtype=x.dtype), batch.astype(jnp.int32),
            num_segments=B)
        return s / jnp.maximum(c, 1.0)[:, None]

    p_user = pool(h_user2, batch_user)
    p_item = pool(h_item2, batch_item)
    x_pool = jnp.concatenate([p_user, p_item], axis=1)
    x_pool = jax.nn.relu(x_pool @ W_lin1 + b_lin1)
    logits = x_pool @ W_lin2 + b_lin2
    return jax.nn.log_softmax(logits, axis=1)
